# Initial kernel scaffold; baseline (speedup 1.0000x reference)
import jax
import jax.numpy as jnp
from jax.experimental import pallas as pl


def kernel(hidden_states):
    raise NotImplementedError("write your pallas kernel here")


# speedup vs baseline: 5.1068x; 5.1068x over previous
import functools
import math

import jax
import jax.numpy as jnp
import numpy as np
from jax.experimental import pallas as pl

STATE_DIM = 1024
VMD_MODES = 4
PERM_DIM = 3
NUM_GROUPS = 16
T = 4096
N1 = 64
B = 4
NPAIR = B * NUM_GROUPS
P6 = math.factorial(PERM_DIM)

_HIGH = jax.lax.Precision.HIGHEST


def _build_constants():
    M = np.zeros((STATE_DIM, NUM_GROUPS), np.float32)
    for g in range(NUM_GROUPS):
        M[g * 64:(g + 1) * 64, g] = 1.0 / 64.0
    idx = np.arange(N1)
    om = np.exp(-2j * np.pi / N1) ** np.outer(idx, idx)
    tw = np.exp(-2j * np.pi / T) ** np.outer(idx, idx)
    freqs = np.fft.fftfreq(T)
    bw = 1.0 / VMD_MODES
    centers = np.linspace(-0.5, 0.5, VMD_MODES)
    filt = np.exp(-0.5 * (np.abs(freqs[None, :] - centers[:, None]) / bw) ** 2)
    filt_pq = filt.reshape(VMD_MODES, N1, N1).transpose(0, 2, 1)
    return dict(
        M=M,
        Fre=om.real.astype(np.float32),
        Fim=om.imag.astype(np.float32),
        TWre=tw.real.astype(np.float32),
        TWim=tw.imag.astype(np.float32),
        filt=filt_pq.astype(np.float32),
    )


_CONSTS = _build_constants()


def _reduce_kernel(h_ref, m_ref, out_ref):
    h = h_ref[0]
    s = jnp.dot(h, m_ref[...], preferred_element_type=jnp.float32,
                precision=_HIGH)
    out_ref[...] = s.reshape(8, N1, NUM_GROUPS).swapaxes(1, 2)


def _main_kernel(s_ref, fre_ref, fim_ref, twre_ref, twim_ref, filt_ref,
                 tm_ref, fm_ref):
    n = NUM_GROUPS
    Fre = fre_ref[...]
    Fim = fim_ref[...]
    TWre = twre_ref[...][:, None, :]
    TWim = twim_ref[...][:, None, :]

    def mm(a, b, dn=None):
        if dn is None:
            return jnp.dot(a, b, preferred_element_type=jnp.float32,
                           precision=_HIGH)
        return jax.lax.dot_general(a, b, dimension_numbers=(dn, ((), ())),
                                   preferred_element_type=jnp.float32,
                                   precision=_HIGH)

    X2 = s_ref[...].reshape(N1, n * N1)
    Gre = mm(Fre, X2).reshape(N1, n, N1)
    Gim = mm(Fim, X2).reshape(N1, n, N1)
    Gpre = Gre * TWre - Gim * TWim
    Gpim = Gre * TWim + Gim * TWre
    Gp2re = Gpre.reshape(N1 * n, N1)
    Gp2im = Gpim.reshape(N1 * n, N1)
    Hre = (mm(Gp2re, Fre) - mm(Gp2im, Fim)).reshape(N1, n, N1)
    Him = (mm(Gp2re, Fim) + mm(Gp2im, Fre)).reshape(N1, n, N1)

    modes = []
    for k in range(VMD_MODES):
        fk = filt_ref[k][:, None, :]
        Hk2re = (Hre * fk).reshape(N1 * n, N1)
        Hk2im = (Him * fk).reshape(N1 * n, N1)
        Ure = (mm(Hk2re, Fre) + mm(Hk2im, Fim)).reshape(N1, n, N1)
        Uim = (mm(Hk2im, Fre) - mm(Hk2re, Fim)).reshape(N1, n, N1)
        Upre = (Ure * TWre + Uim * TWim).reshape(N1, n * N1)
        Upim = (Uim * TWre - Ure * TWim).reshape(N1, n * N1)
        V = (mm(Upre, Fre, dn=((0,), (0,))) +
             mm(Upim, Fim, dn=((0,), (0,)))) * (1.0 / T)
        mk = V.reshape(n, N1, N1).swapaxes(1, 2).reshape(n, T)
        modes.append(mk)

    inds = []
    for k in range(VMD_MODES):
        m = modes[k]
        x0 = m[:, 0:T - 2]
        x1 = m[:, 1:T - 1]
        x2 = m[:, 2:T]
        a = jnp.where(x1 < x0, 1.0, 0.0)
        b = jnp.where(x2 < x0, 1.0, 0.0)
        c = jnp.where(x2 < x1, 1.0, 0.0)
        pk = 2.0 * a + 3.0 * b + c - 2.0 * a * b + a * c
        inds.append([jnp.where(pk == v, 1.0, 0.0) for v in range(P6)])

    iota36 = jax.lax.broadcasted_iota(jnp.int32, (n, P6 * P6), 1)
    hist = jnp.zeros((n, P6 * P6), jnp.float32)
    for va in range(P6):
        for vb in range(P6):
            cnt = jnp.zeros((n,), jnp.float32)
            for k in range(VMD_MODES):
                cnt += jnp.sum(inds[k][va][:, :-1] * inds[k][vb][:, 1:],
                               axis=1)
            hist += jnp.where(iota36 == va * P6 + vb, cnt[:, None], 0.0)
    rowsum = jnp.clip(jnp.sum(hist, axis=1, keepdims=True), 1.0, None)
    tm_ref[...] = hist / rowsum

    ne = []
    for k in range(VMD_MODES):
        e = modes[k] * modes[k]
        mu = jnp.mean(e, axis=1, keepdims=True)
        d = e - mu
        sd = jnp.clip(jnp.sqrt(jnp.sum(d * d, axis=1, keepdims=True)
                               / (T - 1)), 1e-8, None)
        ne.append(d / sd)
    iota6 = jax.lax.broadcasted_iota(jnp.int32, (n, P6), 1)
    fm = jnp.zeros((n, P6), jnp.float32)
    for pidx, (i, j) in enumerate([(0, 1), (0, 2), (0, 3),
                                   (1, 2), (1, 3), (2, 3)]):
        s = jnp.sum(ne[i] * ne[j], axis=1) * (1.0 / T)
        fm += jnp.where(iota6 == pidx, s[:, None], 0.0)
    fm_ref[...] = fm


@functools.partial(jax.jit, static_argnames=("interpret",))
def _run(hidden_states, interpret=False):
    c = _CONSTS
    s3 = pl.pallas_call(
        _reduce_kernel,
        grid=(B, 8),
        in_specs=[
            pl.BlockSpec((1, 512, STATE_DIM), lambda b, i: (b, i, 0)),
            pl.BlockSpec((STATE_DIM, NUM_GROUPS), lambda b, i: (0, 0)),
        ],
        out_specs=pl.BlockSpec((8, NUM_GROUPS, N1), lambda b, i: (i, b, 0)),
        out_shape=jax.ShapeDtypeStruct((N1, NPAIR, N1), jnp.float32),
        interpret=interpret,
    )(hidden_states, c["M"])

    tm, fm = pl.pallas_call(
        _main_kernel,
        grid=(B,),
        in_specs=[
            pl.BlockSpec((N1, NUM_GROUPS, N1), lambda b: (0, b, 0)),
            pl.BlockSpec((N1, N1), lambda b: (0, 0)),
            pl.BlockSpec((N1, N1), lambda b: (0, 0)),
            pl.BlockSpec((N1, N1), lambda b: (0, 0)),
            pl.BlockSpec((N1, N1), lambda b: (0, 0)),
            pl.BlockSpec((VMD_MODES, N1, N1), lambda b: (0, 0, 0)),
        ],
        out_specs=[
            pl.BlockSpec((NUM_GROUPS, P6 * P6), lambda b: (b, 0)),
            pl.BlockSpec((NUM_GROUPS, P6), lambda b: (b, 0)),
        ],
        out_shape=[
            jax.ShapeDtypeStruct((NPAIR, P6 * P6), jnp.float32),
            jax.ShapeDtypeStruct((NPAIR, P6), jnp.float32),
        ],
        interpret=interpret,
    )(s3, c["Fre"], c["Fim"], c["TWre"], c["TWim"], c["filt"])
    return (tm.reshape(B, NUM_GROUPS, P6 * P6),
            fm.reshape(B, NUM_GROUPS, P6))


def kernel(hidden_states):
    return _run(hidden_states)


# speedup vs baseline: 11.3433x; 2.2212x over previous
import functools
import math

import jax
import jax.numpy as jnp
import numpy as np
from jax.experimental import pallas as pl

STATE_DIM = 1024
VMD_MODES = 4
PERM_DIM = 3
NUM_GROUPS = 16
T = 4096
N1 = 64
B = 4
NPAIR = B * NUM_GROUPS
P6 = math.factorial(PERM_DIM)

_HIGH = jax.lax.Precision.DEFAULT


def _build_constants():
    M = np.zeros((STATE_DIM, NUM_GROUPS), np.float32)
    for g in range(NUM_GROUPS):
        M[g * 64:(g + 1) * 64, g] = 1.0 / 64.0
    idx = np.arange(N1)
    om = np.exp(-2j * np.pi / N1) ** np.outer(idx, idx)
    tw = np.exp(-2j * np.pi / T) ** np.outer(idx, idx)
    freqs = np.fft.fftfreq(T)
    bw = 1.0 / VMD_MODES
    centers = np.linspace(-0.5, 0.5, VMD_MODES)
    filt = np.exp(-0.5 * (np.abs(freqs[None, :] - centers[:, None]) / bw) ** 2)
    filt_pq = filt.reshape(VMD_MODES, N1, N1).transpose(0, 2, 1)
    return dict(
        M=M,
        Fre=om.real.astype(np.float32),
        Fim=om.imag.astype(np.float32),
        TWre=tw.real.astype(np.float32),
        TWim=tw.imag.astype(np.float32),
        filt=filt_pq.astype(np.float32),
    )


_CONSTS = _build_constants()


def _reduce_kernel(h_ref, m_ref, out_ref):
    h = h_ref[0]
    s = jnp.dot(h, m_ref[...], preferred_element_type=jnp.float32,
                precision=_HIGH)
    out_ref[...] = s.reshape(16, N1, NUM_GROUPS).swapaxes(1, 2)


def _main_kernel(s_ref, fre_ref, fim_ref, twre_ref, twim_ref, filt_ref,
                 tm_ref, fm_ref):
    n = NUM_GROUPS
    Fre = fre_ref[...]
    Fim = fim_ref[...]
    TWre = twre_ref[...][:, None, :]
    TWim = twim_ref[...][:, None, :]

    def mm(a, b, dn=None):
        if dn is None:
            return jnp.dot(a, b, preferred_element_type=jnp.float32,
                           precision=_HIGH)
        return jax.lax.dot_general(a, b, dimension_numbers=(dn, ((), ())),
                                   preferred_element_type=jnp.float32,
                                   precision=_HIGH)

    X2 = s_ref[...].reshape(N1, n * N1)
    Gre = mm(Fre, X2).reshape(N1, n, N1)
    Gim = mm(Fim, X2).reshape(N1, n, N1)
    Gpre = Gre * TWre - Gim * TWim
    Gpim = Gre * TWim + Gim * TWre
    Gp2re = Gpre.reshape(N1 * n, N1)
    Gp2im = Gpim.reshape(N1 * n, N1)
    Hre = (mm(Gp2re, Fre) - mm(Gp2im, Fim)).reshape(N1, n, N1)
    Him = (mm(Gp2re, Fim) + mm(Gp2im, Fre)).reshape(N1, n, N1)

    modes = []
    for k in range(VMD_MODES):
        fk = filt_ref[k][:, None, :]
        Hk2re = (Hre * fk).reshape(N1 * n, N1)
        Hk2im = (Him * fk).reshape(N1 * n, N1)
        Ure = (mm(Hk2re, Fre) + mm(Hk2im, Fim)).reshape(N1, n, N1)
        Uim = (mm(Hk2im, Fre) - mm(Hk2re, Fim)).reshape(N1, n, N1)
        Upre = (Ure * TWre + Uim * TWim).reshape(N1, n * N1)
        Upim = (Uim * TWre - Ure * TWim).reshape(N1, n * N1)
        V = (mm(Upre, Fre, dn=((0,), (0,))) +
             mm(Upim, Fim, dn=((0,), (0,)))) * (1.0 / T)
        mk = V.reshape(n, N1, N1).swapaxes(1, 2).reshape(n, T)
        modes.append(mk)

    hist = jnp.zeros((n, P6, P6), jnp.float32)
    for k in range(VMD_MODES):
        m = modes[k]
        x0 = m[:, 0:T - 2]
        x1 = m[:, 1:T - 1]
        x2 = m[:, 2:T]
        a = jnp.where(x1 < x0, 1.0, 0.0)
        b = jnp.where(x2 < x0, 1.0, 0.0)
        c = jnp.where(x2 < x1, 1.0, 0.0)
        pk = 2.0 * a + 3.0 * b + c - 2.0 * a * b + a * c
        ind = jnp.concatenate(
            [jnp.where(pk == v, 1.0, 0.0)[:, None, :] for v in range(P6)],
            axis=1).astype(jnp.bfloat16)
        hist += jax.lax.dot_general(
            ind[:, :, :-1], ind[:, :, 1:],
            dimension_numbers=(((2,), (2,)), ((0,), (0,))),
            preferred_element_type=jnp.float32)
    hist = hist.reshape(n, P6 * P6)
    rowsum = jnp.clip(jnp.sum(hist, axis=1, keepdims=True), 1.0, None)
    tm_ref[...] = hist / rowsum

    ne = []
    for k in range(VMD_MODES):
        e = modes[k] * modes[k]
        mu = jnp.mean(e, axis=1, keepdims=True)
        d = e - mu
        sd = jnp.clip(jnp.sqrt(jnp.sum(d * d, axis=1, keepdims=True)
                               / (T - 1)), 1e-8, None)
        ne.append(d / sd)
    iota6 = jax.lax.broadcasted_iota(jnp.int32, (n, P6), 1)
    fm = jnp.zeros((n, P6), jnp.float32)
    for pidx, (i, j) in enumerate([(0, 1), (0, 2), (0, 3),
                                   (1, 2), (1, 3), (2, 3)]):
        s = jnp.sum(ne[i] * ne[j], axis=1) * (1.0 / T)
        fm += jnp.where(iota6 == pidx, s[:, None], 0.0)
    fm_ref[...] = fm


@functools.partial(jax.jit, static_argnames=("interpret",))
def _run(hidden_states, interpret=False):
    c = _CONSTS
    s3 = pl.pallas_call(
        _reduce_kernel,
        grid=(B, 4),
        in_specs=[
            pl.BlockSpec((1, 1024, STATE_DIM), lambda b, i: (b, i, 0)),
            pl.BlockSpec((STATE_DIM, NUM_GROUPS), lambda b, i: (0, 0)),
        ],
        out_specs=pl.BlockSpec((16, NUM_GROUPS, N1), lambda b, i: (i, b, 0)),
        out_shape=jax.ShapeDtypeStruct((N1, NPAIR, N1), jnp.float32),
        interpret=interpret,
    )(hidden_states, c["M"])

    tm, fm = pl.pallas_call(
        _main_kernel,
        grid=(B,),
        in_specs=[
            pl.BlockSpec((N1, NUM_GROUPS, N1), lambda b: (0, b, 0)),
            pl.BlockSpec((N1, N1), lambda b: (0, 0)),
            pl.BlockSpec((N1, N1), lambda b: (0, 0)),
            pl.BlockSpec((N1, N1), lambda b: (0, 0)),
            pl.BlockSpec((N1, N1), lambda b: (0, 0)),
            pl.BlockSpec((VMD_MODES, N1, N1), lambda b: (0, 0, 0)),
        ],
        out_specs=[
            pl.BlockSpec((NUM_GROUPS, P6 * P6), lambda b: (b, 0)),
            pl.BlockSpec((NUM_GROUPS, P6), lambda b: (b, 0)),
        ],
        out_shape=[
            jax.ShapeDtypeStruct((NPAIR, P6 * P6), jnp.float32),
            jax.ShapeDtypeStruct((NPAIR, P6), jnp.float32),
        ],
        interpret=interpret,
    )(s3, c["Fre"], c["Fim"], c["TWre"], c["TWim"], c["filt"])
    return (tm.reshape(B, NUM_GROUPS, P6 * P6),
            fm.reshape(B, NUM_GROUPS, P6))


def kernel(hidden_states):
    return _run(hidden_states)
